# Initial kernel scaffold; baseline (speedup 1.0000x reference)
import jax
import jax.numpy as jnp
from jax.experimental import pallas as pl


def kernel(feats, segment_ids):
    raise NotImplementedError("write your pallas kernel here")


# speedup vs baseline: 3.7820x; 3.7820x over previous
import functools

import jax
import jax.numpy as jnp
from jax import lax
from jax.experimental import pallas as pl
from jax.experimental.pallas import tpu as pltpu
from jax.experimental.pallas import tpu_sc as plsc

_S = 10000
_NW = 32
_B = 400
_ACC_W = 640
_ZB = 8


def _sc_segment_acc(feats, ids):
    n, d = feats.shape
    chunk = n // _NW
    mesh = plsc.VectorSubcoreMesh(core_axis_name="c", subcore_axis_name="s")

    @functools.partial(
        pl.kernel,
        out_type=jax.ShapeDtypeStruct((_S * _ACC_W,), jnp.float32),
        mesh=mesh,
        scratch_types=[
            pltpu.VMEM((_B + 16,), jnp.int32),
            pltpu.VMEM((_B, d), jnp.float32),
            pltpu.VMEM((_ACC_W,), jnp.float32),
            pltpu.VMEM((_ZB * _ACC_W,), jnp.float32),
            pltpu.VMEM((16,), jnp.int32),
            pltpu.SMEM((8,), jnp.int32),
        ],
    )
    def sc_kernel(feats_hbm, ids_hbm, acc_hbm, ids_v, fv, accv, zv, bv, st_s):
        w = lax.axis_index("c") * 16 + lax.axis_index("s")
        cs = w * chunk
        ce = cs + chunk

        @pl.loop(0, _ZB * _ACC_W, step=16)
        def _(i):
            zv[pl.ds(i, 16)] = jnp.zeros((16,), jnp.float32)

        pltpu.sync_copy(
            ids_hbm.at[pl.ds(pl.multiple_of(jnp.maximum(cs - 16, 0), 8), 16)],
            bv)
        prev = jnp.where(w > 0, bv[pl.ds(0, 16)][15], -1)
        pltpu.sync_copy(ids_hbm.at[pl.ds(pl.multiple_of(ce - 16, 8), 16)], bv)
        hi = jnp.where(w < _NW - 1, bv[pl.ds(0, 16)][15] + 1, _S)
        lo = prev + 1

        lo8 = ((lo + _ZB - 1) // _ZB) * _ZB
        hi8 = (hi // _ZB) * _ZB

        def zero_row(z):
            pltpu.sync_copy(
                zv.at[pl.ds(0, _ACC_W)],
                acc_hbm.at[pl.ds(pl.multiple_of(z * _ACC_W, 8), _ACC_W)])

        for t in range(_ZB - 1):
            @pl.when(lo + t < jnp.minimum(hi, lo8))
            def _(t=t):
                zero_row(lo + t)

        for t in range(_ZB - 1):
            @pl.when((hi8 + t >= lo) & (hi8 + t < hi))
            def _(t=t):
                zero_row(hi8 + t)

        @pl.loop(0, _S // _ZB)
        def _(b):
            z = b * _ZB

            @pl.when((z >= lo8) & (z < hi8))
            def _():
                pltpu.sync_copy(
                    zv,
                    acc_hbm.at[pl.ds(pl.multiple_of(z * _ACC_W, 8),
                                     _ZB * _ACC_W)])

        def write_acc(cur, cnt):
            accv[pl.ds(512, 16)] = jnp.full((16,), cnt.astype(jnp.float32))
            accv[pl.ds(528, 16)] = jnp.zeros((16,), jnp.float32)
            pltpu.sync_copy(
                accv,
                acc_hbm.at[pl.ds(pl.multiple_of(cur * _ACC_W, 8), _ACC_W)])

        st_s[0] = jnp.int32(-1)
        st_s[1] = jnp.int32(0)
        st_s[3] = jnp.int32(0)

        @pl.loop(0, n // _B)
        def _(k):
            r = cs + k * _B

            @pl.when((st_s[3] == 0) & (r < n))
            def _():
                pltpu.sync_copy(
                    ids_hbm.at[pl.ds(pl.multiple_of(r, 8), _B)],
                    ids_v.at[pl.ds(0, _B)])
                pltpu.sync_copy(
                    feats_hbm.at[pl.ds(pl.multiple_of(r, 8), _B)], fv)

                @pl.loop(0, _B)
                def _(i):
                    sid = ids_v[pl.ds(i, 16)][0]
                    cur = st_s[0]
                    cnt = st_s[1]
                    live = st_s[3] == 0
                    valid = live & (sid < hi) & (sid >= lo)
                    is_new = valid & (sid != cur)

                    @pl.when(live & (sid >= hi))
                    def _():
                        st_s[3] = jnp.int32(1)

                    @pl.when(is_new)
                    def _():
                        @pl.when(cnt > 0)
                        def _():
                            write_acc(cur, cnt)

                        for j in range(d // 16):
                            x = fv[i, pl.ds(j * 16, 16)]
                            accv[pl.ds(j * 16, 16)] = x
                            accv[pl.ds(128 + j * 16, 16)] = x * x
                            accv[pl.ds(256 + j * 16, 16)] = x
                            accv[pl.ds(384 + j * 16, 16)] = x
                        st_s[0] = sid
                        st_s[1] = jnp.int32(1)

                    @pl.when(valid & jnp.logical_not(is_new))
                    def _():
                        for j in range(d // 16):
                            x = fv[i, pl.ds(j * 16, 16)]
                            accv[pl.ds(j * 16, 16)] += x
                            accv[pl.ds(128 + j * 16, 16)] += x * x
                            accv[pl.ds(256 + j * 16, 16)] = jnp.maximum(
                                accv[pl.ds(256 + j * 16, 16)], x)
                            accv[pl.ds(384 + j * 16, 16)] = jnp.minimum(
                                accv[pl.ds(384 + j * 16, 16)], x)
                        st_s[1] = cnt + 1

        @pl.when(st_s[1] > 0)
        def _():
            write_acc(st_s[0], st_s[1])

    return sc_kernel(feats, ids)


def _tc_finalize(acc):
    bs = 400

    def body(acc_ref, out_ref):
        a = acc_ref[...]
        sm = a[:, 0:128]
        sq = a[:, 128:256]
        mx = a[:, 256:384]
        mn = a[:, 384:512]
        cnt = a[:, 512:513]
        c1 = jnp.maximum(cnt, 1.0)
        mean = sm / c1
        var = (sq - cnt * mean * mean) / jnp.maximum(cnt - 1.0, 1.0)
        std = jnp.sqrt(jnp.clip(var, 0.0) + 1e-12)
        pos = cnt > 0.0
        out_ref[:, 0:128] = mean
        out_ref[:, 128:256] = std
        out_ref[:, 256:384] = jnp.where(pos, mx, 0.0)
        out_ref[:, 384:512] = jnp.where(pos, mn, 0.0)

    return pl.pallas_call(
        body,
        grid=(_S // bs,),
        in_specs=[pl.BlockSpec((bs, _ACC_W), lambda i: (i, 0))],
        out_specs=pl.BlockSpec((bs, 512), lambda i: (i, 0)),
        out_shape=jax.ShapeDtypeStruct((_S, 512), jnp.float32),
    )(acc)


def kernel(feats, segment_ids):
    ids = segment_ids.astype(jnp.int32)
    acc = _sc_segment_acc(feats, ids)
    return _tc_finalize(acc.reshape(_S, _ACC_W))
